# baseline (speedup 1.0000x reference)
import math

import jax
import jax.numpy as jnp
from jax.experimental import pallas as pl
from jax.experimental.pallas import tpu as pltpu

B = 2
T = 4096
E = 2048
D = 128
NT = 8
TB = T // NT
NSEL = int(0.1 * T)
S = 512
_F32_INF_BITS = 0x7F800000


def _body(idx_ref, w_ref, out_ref, qkv_s, nrm_s, pos_s):
    t = pl.program_id(1)
    xb = idx_ref[0].astype(jnp.bfloat16)
    qkv = jnp.dot(xb, w_ref[...], preferred_element_type=jnp.float32)
    qkv_s[pl.ds(t * TB, TB), :] = qkv
    q = qkv[:, :D]
    ones_row = jnp.ones((1, D), jnp.float32)
    n2 = jax.lax.dot_general(ones_row, q * q, (((1,), (1,)), ((), ())),
                             preferred_element_type=jnp.float32, precision=jax.lax.Precision.HIGHEST)
    nrm_s[pl.ds(t, 1), :] = jnp.sqrt(n2)

    @pl.when(t == NT - 1)
    def _finish():
        norms = nrm_s[...]
        bits = jax.lax.bitcast_convert_type(norms, jnp.int32)

        def bs_body(_, carry):
            lo, hi = carry
            mid = lo + (hi - lo) // 2
            cnt = jnp.sum((bits > mid).astype(jnp.int32))
            big = cnt >= NSEL
            return jnp.where(big, mid, lo), jnp.where(big, hi, mid)

        lo, hi = jax.lax.fori_loop(
            0, 31, bs_body, (jnp.int32(-1), jnp.int32(_F32_INF_BITS)))
        thr = hi
        m_gt = bits > thr
        m_eq = bits == thr

        io_i = jax.lax.broadcasted_iota(jnp.int32, (TB, TB), 0)
        io_j = jax.lax.broadcasted_iota(jnp.int32, (TB, TB), 1)
        tri_l = (io_i <= io_j).astype(jnp.float32)
        ro_i = jax.lax.broadcasted_iota(jnp.int32, (NT, NT), 0)
        ro_j = jax.lax.broadcasted_iota(jnp.int32, (NT, NT), 1)
        tri_s = (ro_j < ro_i).astype(jnp.float32)

        def csum(mb):
            mf = mb.astype(jnp.float32)
            within = jnp.dot(mf, tri_l, preferred_element_type=jnp.float32, precision=jax.lax.Precision.HIGHEST)
            off = jnp.dot(tri_s, within[:, TB - 1:TB],
                          preferred_element_type=jnp.float32, precision=jax.lax.Precision.HIGHEST)
            return within + off

        n_gt = jnp.sum(m_gt.astype(jnp.int32))
        need = (NSEL - n_gt).astype(jnp.float32)
        sel = m_gt | (m_eq & (csum(m_eq) <= need))
        pos = csum(sel) - 1.0
        pos_s[...] = jnp.where(sel, pos.astype(jnp.int32), -1)

        iota_s = jax.lax.broadcasted_iota(jnp.int32, (S, TB), 0)

        def build_gather(r, acc):
            pr = pos_s[pl.ds(r, 1), :]
            chunk = (pr == iota_s).astype(jnp.float32)
            return acc + jnp.dot(chunk, qkv_s[pl.ds(r * TB, TB), :],
                                 preferred_element_type=jnp.float32)

        gg = jax.lax.fori_loop(0, NT, build_gather,
                               jnp.zeros((S, 3 * D), jnp.float32))
        qg, kg, vg = gg[:, :D], gg[:, D:2 * D], gg[:, 2 * D:]

        w = jax.lax.dot_general(qg, kg, (((1,), (1,)), ((), ())),
                                preferred_element_type=jnp.float32)
        w = w * (1.0 / math.sqrt(D))
        colmask = jax.lax.broadcasted_iota(jnp.int32, (S, S), 1) < NSEL
        w = jnp.where(colmask, w, -1e30)
        w = w - jnp.max(w, axis=1, keepdims=True)
        p = jnp.exp(w)
        a = p / jnp.sum(p, axis=1, keepdims=True)
        og = jnp.dot(a, vg, preferred_element_type=jnp.float32)

        def scatter(r, _):
            pr = pos_s[pl.ds(r, 1), :]
            chunk = (pr == iota_s).astype(jnp.float32)
            out_ref[0, pl.ds(r * TB, TB), :] = jax.lax.dot_general(
                chunk, og, (((0,), (0,)), ((), ())),
                preferred_element_type=jnp.float32)
            return 0

        jax.lax.fori_loop(0, NT, scatter, 0)


def _run(index, W):
    return pl.pallas_call(
        _body,
        grid=(B, NT),
        in_specs=[
            pl.BlockSpec((1, TB, E), lambda b, t: (b, t, 0)),
            pl.BlockSpec((E, 3 * D), lambda b, t: (0, 0)),
        ],
        out_specs=pl.BlockSpec((1, T, D), lambda b, t: (b, 0, 0)),
        out_shape=jax.ShapeDtypeStruct((B, T, D), jnp.float32),
        scratch_shapes=[
            pltpu.VMEM((T, 3 * D), jnp.float32),
            pltpu.VMEM((NT, TB), jnp.float32),
            pltpu.VMEM((NT, TB), jnp.int32),
        ],
    )(index, W)


def kernel(index, Wq, Wk, Wv):
    W = jnp.concatenate([Wq, Wk, Wv], axis=1).astype(jnp.bfloat16)
    return _run(index, W)
